# baseline (speedup 1.0000x reference)
import functools

import jax
import jax.numpy as jnp
from jax import lax
from jax.experimental import pallas as pl
from jax.experimental.pallas import tpu as pltpu
from jax.experimental.pallas import tpu_sc as plsc

N = 10000
E = 320000
C = 128
NC = 2
NS = 16
NW = NC * NS
EW = E // NW
KA = 80
NCA = EW // KA
NP = 10240
TP = NP // 10


def _sc_mesh():
    return plsc.VectorSubcoreMesh(core_axis_name="c", subcore_axis_name="s")


@functools.partial(
    pl.kernel,
    out_type=jax.ShapeDtypeStruct((NC * NP,), jnp.float32),
    mesh=_sc_mesh(),
    compiler_params=pltpu.CompilerParams(use_tc_tiling_on_sc=False, needs_layout_passes=False),
    scratch_types=[
        pltpu.VMEM((NCA, KA), jnp.int32),
        pltpu.VMEM((KA,), jnp.float32),
        pltpu.VMEM((TP,), jnp.float32),
        pltpu.VMEM_SHARED((NP,), jnp.float32),
        pltpu.SemaphoreType.DMA,
    ],
)
def _deg_sc(edges_hbm, zeros_hbm, out_hbm, dst_v, ones_v, bnc, acc, sem):
    cid = lax.axis_index("c")
    sid = lax.axis_index("s")
    wid = cid * NS + sid
    pltpu.sync_copy(edges_hbm.at[1, wid], dst_v)
    one = jnp.full((16,), 1.0, jnp.float32)
    for j in range(KA // 16):
        ones_v[pl.ds(j * 16, 16)] = one

    @pl.when(sid < 10)
    def _zero():
        pltpu.sync_copy(zeros_hbm, bnc)
        pltpu.sync_copy(bnc, acc.at[pl.ds(sid * TP, TP)])

    plsc.subcore_barrier()

    def fire(g, carry):
        pltpu.async_copy(ones_v, acc.at[dst_v.at[g]], sem, add=True)
        return carry

    lax.fori_loop(0, NCA, fire, 0)

    def drain(g, carry):
        pltpu.make_async_copy(ones_v, acc.at[dst_v.at[g]], sem).wait()
        return carry

    lax.fori_loop(0, NCA, drain, 0)
    plsc.subcore_barrier()

    @pl.when(sid < 10)
    def _write():
        pltpu.sync_copy(acc.at[pl.ds(sid * TP, TP)], bnc)
        pltpu.sync_copy(bnc, out_hbm.at[pl.ds(cid * NP + sid * TP, TP)])


@functools.partial(
    pl.kernel,
    out_type=jax.ShapeDtypeStruct((NC, N, C), jnp.float32),
    mesh=_sc_mesh(),
    compiler_params=pltpu.CompilerParams(use_tc_tiling_on_sc=False, needs_layout_passes=False),
    scratch_types=[
        pltpu.VMEM((NCA, KA), jnp.int32),
        pltpu.VMEM((NCA, KA), jnp.int32),
        pltpu.VMEM((KA, C), jnp.float32),
        pltpu.VMEM((KA, C), jnp.float32),
        pltpu.VMEM((KA, C), jnp.float32),
        pltpu.VMEM_SHARED((N, C), jnp.float32),
        pltpu.SemaphoreType.DMA,
        pltpu.SemaphoreType.DMA,
        pltpu.SemaphoreType.DMA,
        pltpu.SemaphoreType.DMA,
        pltpu.SemaphoreType.DMA,
        pltpu.SemaphoreType.DMA,
    ],
)
def _agg_sc(xs_hbm, edges_hbm, zrows_hbm, out_hbm, src_v, dst_v,
            buf0, buf1, buf2, acc, sem0, sem1, sem2, ssem0, ssem1, ssem2):
    cid = lax.axis_index("c")
    sid = lax.axis_index("s")
    wid = cid * NS + sid
    bufs = (buf0, buf1, buf2)
    sems = (sem0, sem1, sem2)
    ssems = (ssem0, ssem1, ssem2)
    pltpu.sync_copy(edges_hbm.at[0, wid], src_v)
    pltpu.sync_copy(edges_hbm.at[1, wid], dst_v)

    pltpu.async_copy(xs_hbm.at[src_v.at[0]], buf0, sem0)
    pltpu.async_copy(xs_hbm.at[src_v.at[1]], buf1, sem1)

    @pl.when(sid < 10)
    def _zero():
        pltpu.sync_copy(zrows_hbm, buf2.at[pl.ds(0, 40)])
        for j in range(25):
            pltpu.async_copy(buf2.at[pl.ds(0, 40)],
                             acc.at[pl.ds(sid * 1000 + j * 40, 40)], sem2)
        for j in range(25):
            pltpu.make_async_copy(buf2.at[pl.ds(0, 40)],
                                  acc.at[pl.ds(sid * 1000 + j * 40, 40)],
                                  sem2).wait()

    plsc.subcore_barrier()

    pltpu.make_async_copy(xs_hbm.at[src_v.at[0]], buf0, sem0).wait()
    pltpu.sync_copy(buf0, acc.at[dst_v.at[0]], add=True)
    pltpu.async_copy(xs_hbm.at[src_v.at[2]], buf2, sem2)

    def body(t, carry):
        for bp in range(3):
            g = 3 * t + 1 + bp
            b = (1 + bp) % 3
            nb = (b + 2) % 3
            pltpu.make_async_copy(xs_hbm.at[src_v.at[g]], bufs[b], sems[b]).wait()
            if bp == 0:
                @pl.when(t > 0)
                def _dr():
                    pltpu.make_async_copy(bufs[nb], acc.at[pl.ds(0, KA)],
                                          ssems[nb]).wait()
            else:
                pltpu.make_async_copy(bufs[nb], acc.at[pl.ds(0, KA)],
                                      ssems[nb]).wait()

            @pl.when(g < NCA - 2)
            def _gnext():
                pltpu.async_copy(xs_hbm.at[src_v.at[g + 2]], bufs[nb], sems[nb])

            pltpu.async_copy(bufs[b], acc.at[dst_v.at[g]], ssems[b], add=True)
        return carry

    lax.fori_loop(0, (NCA - 2) // 3, body, 0)
    pltpu.make_async_copy(xs_hbm.at[src_v.at[NCA - 1]], buf1, sem1).wait()
    pltpu.async_copy(buf1, acc.at[dst_v.at[NCA - 1]], ssem1, add=True)
    pltpu.make_async_copy(buf0, acc.at[pl.ds(0, KA)], ssem0).wait()
    pltpu.make_async_copy(buf1, acc.at[pl.ds(0, KA)], ssem1).wait()
    plsc.subcore_barrier()

    @pl.when(sid < 10)
    def _write():
        for j in range(25):
            bb, ss = bufs[j % 2], sems[j % 2]
            if j >= 2:
                pltpu.make_async_copy(
                    bb.at[pl.ds(0, 40)],
                    out_hbm.at[cid, pl.ds(sid * 1000 + (j - 2) * 40, 40)],
                    ss).wait()
            pltpu.sync_copy(acc.at[pl.ds(sid * 1000 + j * 40, 40)],
                            bb.at[pl.ds(0, 40)])
            pltpu.async_copy(bb.at[pl.ds(0, 40)],
                             out_hbm.at[cid, pl.ds(sid * 1000 + j * 40, 40)], ss)
        for j in (23, 24):
            bb, ss = bufs[j % 2], sems[j % 2]
            pltpu.make_async_copy(
                bb.at[pl.ds(0, 40)],
                out_hbm.at[cid, pl.ds(sid * 1000 + j * 40, 40)], ss).wait()


@functools.partial(
    pl.kernel,
    out_type=jax.ShapeDtypeStruct((NC * NP,), jnp.float32),
    mesh=_sc_mesh(),
    compiler_params=pltpu.CompilerParams(use_tc_tiling_on_sc=False, needs_layout_passes=False),
    scratch_types=[
        pltpu.VMEM((NCA, KA), jnp.int32),
        pltpu.VMEM((NCA, KA), jnp.int32),
        pltpu.VMEM((N,), jnp.float32),
        pltpu.VMEM((KA,), jnp.float32),
        pltpu.VMEM((KA,), jnp.float32),
        pltpu.VMEM((KA,), jnp.float32),
        pltpu.VMEM((KA,), jnp.float32),
        pltpu.VMEM((TP,), jnp.float32),
        pltpu.VMEM_SHARED((NP,), jnp.float32),
        pltpu.SemaphoreType.DMA,
        pltpu.SemaphoreType.DMA,
        pltpu.SemaphoreType.DMA,
        pltpu.SemaphoreType.DMA,
    ],
)
def _agg1_sc(xs_hbm, edges_hbm, zeros_hbm, out_hbm, src_v, dst_v, xsl,
             buf0, buf1, buf2, buf3, bnc, acc, sem0, sem1, sem2, sem3):
    cid = lax.axis_index("c")
    sid = lax.axis_index("s")
    wid = cid * NS + sid
    bufs = (buf0, buf1, buf2, buf3)
    sems = (sem0, sem1, sem2, sem3)
    pltpu.sync_copy(edges_hbm.at[0, wid], src_v)
    pltpu.sync_copy(edges_hbm.at[1, wid], dst_v)
    pltpu.sync_copy(xs_hbm, xsl)

    @pl.when(sid < 10)
    def _zero():
        pltpu.sync_copy(zeros_hbm, bnc)
        pltpu.sync_copy(bnc, acc.at[pl.ds(sid * TP, TP)])

    plsc.subcore_barrier()

    def _fill(g, b):
        for j in range(KA // 16):
            idx = src_v[g, pl.ds(j * 16, 16)]
            bufs[b][pl.ds(j * 16, 16)] = plsc.load_gather(xsl, [idx])

    def body(t, carry):
        for b in range(4):
            g = 4 * t + b

            @pl.when(t > 0)
            def _drain():
                pltpu.make_async_copy(bufs[b], acc.at[dst_v.at[g]], sems[b]).wait()

            _fill(g, b)
            pltpu.async_copy(bufs[b], acc.at[dst_v.at[g]], sems[b], add=True)
        return carry

    lax.fori_loop(0, (NCA - 1) // 4, body, 0)
    pltpu.make_async_copy(buf0, acc.at[pl.ds(0, KA)], sem0).wait()
    _fill(NCA - 1, 0)
    pltpu.async_copy(buf0, acc.at[dst_v.at[NCA - 1]], sem0, add=True)
    for b in range(4):
        pltpu.make_async_copy(bufs[b], acc.at[pl.ds(0, KA)], sems[b]).wait()
    plsc.subcore_barrier()

    @pl.when(sid < 10)
    def _write():
        pltpu.sync_copy(acc.at[pl.ds(sid * TP, TP)], bnc)
        pltpu.sync_copy(bnc, out_hbm.at[pl.ds(cid * NP + sid * TP, TP)])


def _tc_mm1(x, W1):
    def body(x_ref, w_ref, t_ref):
        t_ref[...] = jnp.dot(x_ref[...], w_ref[...], preferred_element_type=jnp.float32)

    return pl.pallas_call(
        body,
        out_shape=jax.ShapeDtypeStruct((N, C), jnp.float32),
    )(x, W1)


def _tc_scale1(deg_p, t1):
    def body(degp_ref, t_ref, xs_ref, dinv_ref):
        deg = 1.0 + degp_ref[pl.ds(0, N)] + degp_ref[pl.ds(NP, N)]
        dinv = lax.rsqrt(deg)
        dinv_ref[...] = dinv
        xs_ref[...] = dinv[:, None] * t_ref[...]

    return pl.pallas_call(
        body,
        out_shape=[
            jax.ShapeDtypeStruct((N, C), jnp.float32),
            jax.ShapeDtypeStruct((N,), jnp.float32),
        ],
    )(deg_p, t1)


def _tc_mid(acc_p, xs, dinv, b, Wn):
    Cn = Wn.shape[1]

    def body(accp_ref, xs_ref, dinv_ref, b_ref, w_ref, h_ref, xsn_ref):
        s = accp_ref[0] + accp_ref[1] + xs_ref[...]
        dinv = dinv_ref[...]
        h = jnp.maximum(dinv[:, None] * s + b_ref[...], 0.0)
        h_ref[...] = h
        hw = jnp.dot(h, w_ref[...], preferred_element_type=jnp.float32)
        xsn_ref[...] = dinv[:, None] * hw

    return pl.pallas_call(
        body,
        out_shape=[
            jax.ShapeDtypeStruct((N, C), jnp.float32),
            jax.ShapeDtypeStruct((N, Cn), jnp.float32),
        ],
    )(acc_p, xs, dinv, b, Wn)


def _tc_mid3(acc_p, xs, dinv, b, w3):
    def body(accp_ref, xs_ref, dinv_ref, b_ref, w_ref, h_ref, xs3_ref):
        s = accp_ref[0] + accp_ref[1] + xs_ref[...]
        dinv = dinv_ref[...]
        h = jnp.maximum(dinv[:, None] * s + b_ref[...], 0.0)
        h_ref[...] = h
        hw = jnp.dot(h, w_ref[...], preferred_element_type=jnp.float32)
        xs3_ref[...] = dinv * hw

    return pl.pallas_call(
        body,
        out_shape=[
            jax.ShapeDtypeStruct((N, C), jnp.float32),
            jax.ShapeDtypeStruct((N,), jnp.float32),
        ],
    )(acc_p, xs, dinv, b, w3)


def _tc_final(acc_p, xs3, dinv, b3):
    def body(accp_ref, xs_ref, dinv_ref, b_ref, y_ref):
        ap = accp_ref[pl.ds(0, N)] + accp_ref[pl.ds(NP, N)]
        y_ref[...] = dinv_ref[...] * (ap + xs_ref[...]) + b_ref[...]

    return pl.pallas_call(
        body,
        out_shape=jax.ShapeDtypeStruct((N,), jnp.float32),
    )(acc_p, xs3, dinv, b3)


def kernel(x, edge_index, W1, b1, W2, b2, W3, b3):
    e32 = edge_index.astype(jnp.int32)
    edges = e32.reshape(2, NW, NCA, KA)
    zrows = jnp.zeros((40, C), jnp.float32)
    ztp = jnp.zeros((TP,), jnp.float32)

    t1 = _tc_mm1(x, W1)
    deg_p = _deg_sc(edges, ztp)
    xs1, dinv = _tc_scale1(deg_p, t1)
    acc1_p = _agg_sc(xs1, edges, zrows)
    h1, xs2 = _tc_mid(acc1_p, xs1, dinv, b1, W2)
    acc2_p = _agg_sc(xs2, edges, zrows)
    h2, xs3f = _tc_mid3(acc2_p, xs2, dinv, b2, W3.reshape(-1))
    acc3_p = _agg1_sc(xs3f, edges, ztp)
    y = _tc_final(acc3_p, xs3f, dinv, b3)
    return (y, h1, h2)


# speedup vs baseline: 1.0082x; 1.0082x over previous
import functools

import jax
import jax.numpy as jnp
from jax import lax
from jax.experimental import pallas as pl
from jax.experimental.pallas import tpu as pltpu
from jax.experimental.pallas import tpu_sc as plsc

N = 10000
E = 320000
C = 128
NC = 2
NS = 16
NW = NC * NS
EW = E // NW
KA = 80
NCA = EW // KA
NP = 10240
TP = NP // 10


def _sc_mesh():
    return plsc.VectorSubcoreMesh(core_axis_name="c", subcore_axis_name="s")


@functools.partial(
    pl.kernel,
    out_type=jax.ShapeDtypeStruct((NC * NP,), jnp.float32),
    mesh=_sc_mesh(),
    compiler_params=pltpu.CompilerParams(use_tc_tiling_on_sc=False, needs_layout_passes=False),
    scratch_types=[
        pltpu.VMEM((NCA, KA), jnp.int32),
        pltpu.VMEM((KA,), jnp.float32),
        pltpu.VMEM((TP,), jnp.float32),
        pltpu.VMEM_SHARED((NP,), jnp.float32),
        pltpu.SemaphoreType.DMA,
    ],
)
def _deg_sc(edges_hbm, zeros_hbm, out_hbm, dst_v, ones_v, bnc, acc, sem):
    cid = lax.axis_index("c")
    sid = lax.axis_index("s")
    wid = cid * NS + sid
    pltpu.sync_copy(edges_hbm.at[1, wid], dst_v)
    one = jnp.full((16,), 1.0, jnp.float32)
    for j in range(KA // 16):
        ones_v[pl.ds(j * 16, 16)] = one

    @pl.when(sid < 10)
    def _zero():
        pltpu.sync_copy(zeros_hbm, bnc)
        pltpu.sync_copy(bnc, acc.at[pl.ds(sid * TP, TP)])

    plsc.subcore_barrier()

    def fire(g, carry):
        pltpu.async_copy(ones_v, acc.at[dst_v.at[g]], sem, add=True)
        return carry

    lax.fori_loop(0, NCA, fire, 0)

    def drain(g, carry):
        pltpu.make_async_copy(ones_v, acc.at[dst_v.at[g]], sem).wait()
        return carry

    lax.fori_loop(0, NCA, drain, 0)
    plsc.subcore_barrier()

    @pl.when(sid < 10)
    def _write():
        pltpu.sync_copy(acc.at[pl.ds(sid * TP, TP)], bnc)
        pltpu.sync_copy(bnc, out_hbm.at[pl.ds(cid * NP + sid * TP, TP)])


@functools.partial(
    pl.kernel,
    out_type=jax.ShapeDtypeStruct((NC, N, C), jnp.float32),
    mesh=_sc_mesh(),
    compiler_params=pltpu.CompilerParams(use_tc_tiling_on_sc=False, needs_layout_passes=False),
    scratch_types=[
        pltpu.VMEM((NCA, KA), jnp.int32),
        pltpu.VMEM((NCA, KA), jnp.int32),
        pltpu.VMEM((KA, C), jnp.float32),
        pltpu.VMEM((KA, C), jnp.float32),
        pltpu.VMEM((KA, C), jnp.float32),
        pltpu.VMEM_SHARED((N, C), jnp.float32),
        pltpu.SemaphoreType.DMA,
        pltpu.SemaphoreType.DMA,
        pltpu.SemaphoreType.DMA,
    ],
)
def _agg_sc(xs_hbm, edges_hbm, zrows_hbm, out_hbm, src_v, dst_v,
            buf0, buf1, buf2, acc, sem0, sem1, sem2):
    cid = lax.axis_index("c")
    sid = lax.axis_index("s")
    wid = cid * NS + sid
    bufs = (buf0, buf1, buf2)
    sems = (sem0, sem1, sem2)
    pltpu.sync_copy(edges_hbm.at[0, wid], src_v)
    pltpu.sync_copy(edges_hbm.at[1, wid], dst_v)

    pltpu.async_copy(xs_hbm.at[src_v.at[0]], buf0, sem0)
    pltpu.async_copy(xs_hbm.at[src_v.at[1]], buf1, sem1)

    @pl.when(sid < 10)
    def _zero():
        pltpu.sync_copy(zrows_hbm, buf2.at[pl.ds(0, 40)])
        for j in range(25):
            pltpu.async_copy(buf2.at[pl.ds(0, 40)],
                             acc.at[pl.ds(sid * 1000 + j * 40, 40)], sem2)
        for j in range(25):
            pltpu.make_async_copy(buf2.at[pl.ds(0, 40)],
                                  acc.at[pl.ds(sid * 1000 + j * 40, 40)],
                                  sem2).wait()

    plsc.subcore_barrier()

    def body(t, carry):
        g0 = 3 * t
        for b in range(3):
            g = g0 + b
            nb = (b + 2) % 3
            pltpu.make_async_copy(xs_hbm.at[src_v.at[g]], bufs[b], sems[b]).wait()
            pltpu.async_copy(xs_hbm.at[src_v.at[g + 2]], bufs[nb], sems[nb])
            pltpu.sync_copy(bufs[b], acc.at[dst_v.at[g]], add=True)
        return carry

    lax.fori_loop(0, (NCA - 2) // 3, body, 0)
    for g in (NCA - 2, NCA - 1):
        b = g % 3
        pltpu.make_async_copy(xs_hbm.at[src_v.at[g]], bufs[b], sems[b]).wait()
        pltpu.sync_copy(bufs[b], acc.at[dst_v.at[g]], add=True)
    plsc.subcore_barrier()

    @pl.when(sid < 10)
    def _write():
        for j in range(25):
            bb, ss = bufs[j % 2], sems[j % 2]
            if j >= 2:
                pltpu.make_async_copy(
                    bb.at[pl.ds(0, 40)],
                    out_hbm.at[cid, pl.ds(sid * 1000 + (j - 2) * 40, 40)],
                    ss).wait()
            pltpu.sync_copy(acc.at[pl.ds(sid * 1000 + j * 40, 40)],
                            bb.at[pl.ds(0, 40)])
            pltpu.async_copy(bb.at[pl.ds(0, 40)],
                             out_hbm.at[cid, pl.ds(sid * 1000 + j * 40, 40)], ss)
        for j in (23, 24):
            bb, ss = bufs[j % 2], sems[j % 2]
            pltpu.make_async_copy(
                bb.at[pl.ds(0, 40)],
                out_hbm.at[cid, pl.ds(sid * 1000 + j * 40, 40)], ss).wait()


@functools.partial(
    pl.kernel,
    out_type=jax.ShapeDtypeStruct((NC * NP,), jnp.float32),
    mesh=_sc_mesh(),
    compiler_params=pltpu.CompilerParams(use_tc_tiling_on_sc=False, needs_layout_passes=False),
    scratch_types=[
        pltpu.VMEM((NCA, KA), jnp.int32),
        pltpu.VMEM((NCA, KA), jnp.int32),
        pltpu.VMEM((N,), jnp.float32),
        pltpu.VMEM((KA,), jnp.float32),
        pltpu.VMEM((KA,), jnp.float32),
        pltpu.VMEM((KA,), jnp.float32),
        pltpu.VMEM((KA,), jnp.float32),
        pltpu.VMEM((TP,), jnp.float32),
        pltpu.VMEM_SHARED((NP,), jnp.float32),
        pltpu.SemaphoreType.DMA,
        pltpu.SemaphoreType.DMA,
        pltpu.SemaphoreType.DMA,
        pltpu.SemaphoreType.DMA,
    ],
)
def _agg1_sc(xs_hbm, edges_hbm, zeros_hbm, out_hbm, src_v, dst_v, xsl,
             buf0, buf1, buf2, buf3, bnc, acc, sem0, sem1, sem2, sem3):
    cid = lax.axis_index("c")
    sid = lax.axis_index("s")
    wid = cid * NS + sid
    bufs = (buf0, buf1, buf2, buf3)
    sems = (sem0, sem1, sem2, sem3)
    pltpu.sync_copy(edges_hbm.at[0, wid], src_v)
    pltpu.sync_copy(edges_hbm.at[1, wid], dst_v)
    pltpu.sync_copy(xs_hbm, xsl)

    @pl.when(sid < 10)
    def _zero():
        pltpu.sync_copy(zeros_hbm, bnc)
        pltpu.sync_copy(bnc, acc.at[pl.ds(sid * TP, TP)])

    plsc.subcore_barrier()

    def _fill(g, b):
        for j in range(KA // 16):
            idx = src_v[g, pl.ds(j * 16, 16)]
            bufs[b][pl.ds(j * 16, 16)] = plsc.load_gather(xsl, [idx])

    def body(t, carry):
        for b in range(4):
            g = 4 * t + b

            @pl.when(t > 0)
            def _drain():
                pltpu.make_async_copy(bufs[b], acc.at[dst_v.at[g]], sems[b]).wait()

            _fill(g, b)
            pltpu.async_copy(bufs[b], acc.at[dst_v.at[g]], sems[b], add=True)
        return carry

    lax.fori_loop(0, (NCA - 1) // 4, body, 0)
    pltpu.make_async_copy(buf0, acc.at[pl.ds(0, KA)], sem0).wait()
    _fill(NCA - 1, 0)
    pltpu.async_copy(buf0, acc.at[dst_v.at[NCA - 1]], sem0, add=True)
    for b in range(4):
        pltpu.make_async_copy(bufs[b], acc.at[pl.ds(0, KA)], sems[b]).wait()
    plsc.subcore_barrier()

    @pl.when(sid < 10)
    def _write():
        pltpu.sync_copy(acc.at[pl.ds(sid * TP, TP)], bnc)
        pltpu.sync_copy(bnc, out_hbm.at[pl.ds(cid * NP + sid * TP, TP)])


def _tc_mm1(x, W1):
    def body(x_ref, w_ref, t_ref):
        t_ref[...] = jnp.dot(x_ref[...], w_ref[...], preferred_element_type=jnp.float32)

    return pl.pallas_call(
        body,
        out_shape=jax.ShapeDtypeStruct((N, C), jnp.float32),
    )(x, W1)


def _tc_scale1(deg_p, t1):
    def body(degp_ref, t_ref, xs_ref, dinv_ref):
        deg = 1.0 + degp_ref[pl.ds(0, N)] + degp_ref[pl.ds(NP, N)]
        dinv = lax.rsqrt(deg)
        dinv_ref[...] = dinv
        xs_ref[...] = dinv[:, None] * t_ref[...]

    return pl.pallas_call(
        body,
        out_shape=[
            jax.ShapeDtypeStruct((N, C), jnp.float32),
            jax.ShapeDtypeStruct((N,), jnp.float32),
        ],
    )(deg_p, t1)


def _tc_mid(acc_p, xs, dinv, b, Wn):
    Cn = Wn.shape[1]

    def body(accp_ref, xs_ref, dinv_ref, b_ref, w_ref, h_ref, xsn_ref):
        s = accp_ref[0] + accp_ref[1] + xs_ref[...]
        dinv = dinv_ref[...]
        h = jnp.maximum(dinv[:, None] * s + b_ref[...], 0.0)
        h_ref[...] = h
        hw = jnp.dot(h, w_ref[...], preferred_element_type=jnp.float32)
        xsn_ref[...] = dinv[:, None] * hw

    return pl.pallas_call(
        body,
        out_shape=[
            jax.ShapeDtypeStruct((N, C), jnp.float32),
            jax.ShapeDtypeStruct((N, Cn), jnp.float32),
        ],
    )(acc_p, xs, dinv, b, Wn)


def _tc_mid3(acc_p, xs, dinv, b, w3):
    def body(accp_ref, xs_ref, dinv_ref, b_ref, w_ref, h_ref, xs3_ref):
        s = accp_ref[0] + accp_ref[1] + xs_ref[...]
        dinv = dinv_ref[...]
        h = jnp.maximum(dinv[:, None] * s + b_ref[...], 0.0)
        h_ref[...] = h
        hw = jnp.dot(h, w_ref[...], preferred_element_type=jnp.float32)
        xs3_ref[...] = dinv * hw

    return pl.pallas_call(
        body,
        out_shape=[
            jax.ShapeDtypeStruct((N, C), jnp.float32),
            jax.ShapeDtypeStruct((N,), jnp.float32),
        ],
    )(acc_p, xs, dinv, b, w3)


def _tc_final(acc_p, xs3, dinv, b3):
    def body(accp_ref, xs_ref, dinv_ref, b_ref, y_ref):
        ap = accp_ref[pl.ds(0, N)] + accp_ref[pl.ds(NP, N)]
        y_ref[...] = dinv_ref[...] * (ap + xs_ref[...]) + b_ref[...]

    return pl.pallas_call(
        body,
        out_shape=jax.ShapeDtypeStruct((N,), jnp.float32),
    )(acc_p, xs3, dinv, b3)


def kernel(x, edge_index, W1, b1, W2, b2, W3, b3):
    e32 = edge_index.astype(jnp.int32)
    edges = e32.reshape(2, NW, NCA, KA)
    zrows = jnp.zeros((40, C), jnp.float32)
    ztp = jnp.zeros((TP,), jnp.float32)

    t1 = _tc_mm1(x, W1)
    deg_p = _deg_sc(edges, ztp)
    xs1, dinv = _tc_scale1(deg_p, t1)
    acc1_p = _agg_sc(xs1, edges, zrows)
    h1, xs2 = _tc_mid(acc1_p, xs1, dinv, b1, W2)
    acc2_p = _agg_sc(xs2, edges, zrows)
    h2, xs3f = _tc_mid3(acc2_p, xs2, dinv, b2, W3.reshape(-1))
    acc3_p = _agg1_sc(xs3f, edges, ztp)
    y = _tc_final(acc3_p, xs3f, dinv, b3)
    return (y, h1, h2)
